# baseline (speedup 1.0000x reference)
import functools

import jax
import jax.numpy as jnp
from jax import lax
from jax.experimental import pallas as pl
from jax.experimental.pallas import tpu as pltpu
from jax.experimental.pallas import tpu_sc as plsc

B = 4096
L = 200
C = 100001
T1 = 12
BASE = 16
N = B * L

_TW = 2048
_TGRID = -(-C // _TW)
CP = _TGRID * _TW

_info = plsc.get_sparse_core_info()
NC, NS, LANES = _info.num_cores, _info.num_subcores, _info.num_lanes
NW = NC * NS
TOK_PER_W = N // NW
CHUNK = 512
NCHUNK = TOK_PER_W // CHUNK


@functools.partial(
    pl.kernel,
    mesh=plsc.VectorSubcoreMesh(core_axis_name="c", subcore_axis_name="s"),
    out_type=jax.ShapeDtypeStruct((L, 4, B // 128, 8, 128), jnp.float32),
    compiler_params=pltpu.CompilerParams(
        needs_layout_passes=False, use_tc_tiling_on_sc=False
    ),
    scratch_types=[
        pltpu.VMEM((2, 3, CHUNK), jnp.int32),
        pltpu.VMEM((2, 2 * CHUNK), jnp.int32),
        pltpu.VMEM((2, 2 * CHUNK, BASE), jnp.float32),
        pltpu.VMEM((2, 4, 4, 8, 128), jnp.float32),
        pltpu.SemaphoreType.DMA,
        pltpu.SemaphoreType.DMA,
        pltpu.SemaphoreType.DMA,
        pltpu.SemaphoreType.DMA,
        pltpu.SemaphoreType.DMA,
    ],
)
def _sc_gather(table_hbm, t1_hbm, t2_hbm, item_hbm, out_hbm,
               tin, idx2, rows2, obuf, sem_in0, sem_in1, sem_g,
               sem_o0, sem_o1):
    wid = lax.axis_index("s") * NC + lax.axis_index("c")
    lane = lax.iota(jnp.int32, LANES)

    def in_copies(k, buf):
        cid = wid * NCHUNK + k
        pbase = (cid >> 3) * B + (cid & 7) * CHUNK
        sem_in = sem_in1 if buf else sem_in0
        run = pl.ds(pbase, CHUNK)
        return [
            pltpu.make_async_copy(src.at[run], tin.at[buf, ai], sem_in)
            for ai, src in enumerate((t1_hbm, t2_hbm, item_hbm))
        ]

    def out_copies(k, buf):
        cid = wid * NCHUNK + k
        lp = cid >> 3
        bt0 = (cid & 7) * 4
        sem_o = sem_o1 if buf else sem_o0
        return [
            pltpu.make_async_copy(
                obuf.at[buf, ft, btl],
                out_hbm.at[lp, ft, bt0 + btl], sem_o)
            for ft in range(4) for btl in range(4)
        ]

    def gather_copy(buf):
        return pltpu.make_async_copy(
            table_hbm.at[idx2.at[buf]], rows2.at[buf], sem_g)

    def build_idx(buf):
        def vec_body(j, carry2):
            sl = pl.ds(j * LANES, LANES)
            it8 = tin[buf, 2, sl] * 8
            mk = tin[buf, 0, sl]
            wk = tin[buf, 1, sl] + T1
            m_idx = (mk >> 3) * (CP * 8) + it8 + (mk & 7)
            w_idx = (wk >> 3) * (CP * 8) + it8 + (wk & 7)
            pos = lane * 2 + j * (2 * LANES)
            plsc.store_scatter(idx2.at[buf], [pos], m_idx)
            plsc.store_scatter(idx2.at[buf], [pos + 1], w_idx)
            return carry2

        lax.fori_loop(0, CHUNK // LANES, vec_body, 0)

    def relayout(buf):
        def g_body(g, carry2):
            row0 = 32 * g + 2 * lane
            bs = pl.ds((g & 7) * LANES, LANES)
            for f in range(2 * BASE):
                vec = plsc.load_gather(
                    rows2.at[buf],
                    [row0 + (f // BASE),
                     jnp.full((LANES,), f % BASE, jnp.int32)])
                obuf[buf, f // 8, g >> 3, f % 8, bs] = vec
            return carry2

        lax.fori_loop(0, CHUNK // LANES, g_body, 0)

    for h in in_copies(0, 0):
        h.start()

    def loop_body(i, carry):
        for par in range(2):
            k = 2 * i + par
            buf = par
            nbuf = 1 - par

            @pl.when(k + 1 < NCHUNK)
            def _():
                for h in in_copies(k + 1, nbuf):
                    h.start()

            for h in in_copies(k, buf):
                h.wait()

            build_idx(buf)
            @pl.when(k >= 2)
            def _():
                for h in out_copies(k - 2, buf):
                    h.wait()

            gather_copy(buf).start()

            @pl.when(k >= 1)
            def _():
                gather_copy(nbuf).wait()
                relayout(nbuf)
                for h in out_copies(k - 1, nbuf):
                    h.start()

        return carry

    lax.fori_loop(0, NCHUNK // 2, loop_body, 0)
    last = NCHUNK - 1
    gather_copy(last & 1).wait()
    relayout(last & 1)
    for h in out_copies(last, last & 1):
        h.start()
    for h in out_copies(last - 1, 1 - (last & 1)):
        h.wait()
    for h in out_copies(last, last & 1):
        h.wait()


def _tc_transpose_body(m_ref, w_ref, o_ref):
    o_ref[0] = m_ref[0:128, :].T
    o_ref[1] = jnp.concatenate([m_ref[128:192, :], w_ref[0:64, :]], axis=0).T
    o_ref[2, :, 0:16] = w_ref[64:80, :].T


_tc_transpose = pl.pallas_call(
    _tc_transpose_body,
    grid=(_TGRID,),
    in_specs=[
        pl.BlockSpec((192, _TW), lambda p: (0, p)),
        pl.BlockSpec((80, _TW), lambda p: (0, p)),
    ],
    out_specs=pl.BlockSpec((3, _TW, 128), lambda p: (0, p, 0)),
    out_shape=jax.ShapeDtypeStruct((3, CP, 128), jnp.float32),
)


def _tc_relayout_body(x_ref, o_ref):
    for i in range(8):
        z = x_ref[i].T
        for g in range(4):
            o_ref[i, :, pl.ds(g * 1024, 1024)] = z[32 * g:32 * (g + 1), :]


_tc_relayout = pl.pallas_call(
    _tc_relayout_body,
    grid=(L // 8,),
    in_specs=[pl.BlockSpec((8, 1024, 128), lambda l: (l, 0, 0))],
    out_specs=pl.BlockSpec((8, 2 * BASE, B), lambda l: (l, 0, 0)),
    out_shape=jax.ShapeDtypeStruct((L, 2 * BASE, B), jnp.float32),
)


def kernel(log_seqs, time1_seqs, time2_seqs, month_pop_table, week_pop_table):
    table = _tc_transpose(month_pop_table, week_pop_table).reshape(3 * CP * 8, BASE)
    t1 = time1_seqs.T.reshape(-1).astype(jnp.int32)
    t2 = time2_seqs.T.reshape(-1).astype(jnp.int32)
    item = log_seqs.T.reshape(-1).astype(jnp.int32)
    out5 = _sc_gather(table, t1, t2, item)
    return out5.transpose(2, 4, 0, 1, 3).reshape(B, L, 2 * BASE)


# speedup vs baseline: 2.8195x; 2.8195x over previous
import functools

import jax
import jax.numpy as jnp
from jax import lax
from jax.experimental import pallas as pl
from jax.experimental.pallas import tpu as pltpu
from jax.experimental.pallas import tpu_sc as plsc

B = 4096
L = 200
C = 100001
T1 = 12
BASE = 16
N = B * L

_TW = 4096
_TGRID = -(-C // _TW)
CP = _TGRID * _TW

_info = plsc.get_sparse_core_info()
NC, NS, LANES = _info.num_cores, _info.num_subcores, _info.num_lanes
NW = NC * NS
TOK_PER_W = N // NW
CHUNK = 1024
NCHUNK = TOK_PER_W // CHUNK


@functools.partial(
    pl.kernel,
    mesh=plsc.VectorSubcoreMesh(core_axis_name="c", subcore_axis_name="s"),
    out_type=jax.ShapeDtypeStruct((2 * N, BASE), jnp.float32),
    compiler_params=pltpu.CompilerParams(
        needs_layout_passes=False, use_tc_tiling_on_sc=False
    ),
    scratch_types=[
        pltpu.VMEM((2, 3, CHUNK), jnp.int32),
        pltpu.VMEM((2, 2 * CHUNK), jnp.int32),
        pltpu.VMEM((2, 2 * CHUNK, BASE), jnp.float32),
        pltpu.SemaphoreType.DMA,
        pltpu.SemaphoreType.DMA,
        pltpu.SemaphoreType.DMA,
        pltpu.SemaphoreType.DMA,
    ],
)
def _sc_gather(table_hbm, t1_hbm, t2_hbm, item_hbm, out_hbm,
               tin, idx2, rows2, sem_in0, sem_in1, sem_g, sem_out):
    wid = lax.axis_index("s") * NC + lax.axis_index("c")
    lane = lax.iota(jnp.int32, LANES)
    p0 = (lane & 3) * 256 + (lane >> 2)

    def in_copies(k, buf):
        cid = wid * NCHUNK + k
        pbase = (cid >> 2) * B + (cid & 3) * 256
        sem_in = sem_in1 if buf else sem_in0
        out = []
        for g in range(4):
            run = pl.ds(pbase + g * 1024, 256)
            for ai, src in enumerate((t1_hbm, t2_hbm, item_hbm)):
                out.append(pltpu.make_async_copy(
                    src.at[run], tin.at[buf, ai, pl.ds(g * 256, 256)], sem_in))
        return out

    def out_copy(k, buf):
        cid = wid * NCHUNK + k
        return pltpu.make_async_copy(
            rows2.at[buf], out_hbm.at[pl.ds(2 * cid * CHUNK, 2 * CHUNK)],
            sem_out)

    def gather_copy(buf):
        return pltpu.make_async_copy(
            table_hbm.at[idx2.at[buf]], rows2.at[buf], sem_g)

    def build_idx(buf):
        def vec_body(j, carry2):
            pos_in = p0 + 4 * j
            it8 = plsc.load_gather(tin.at[buf, 2], [pos_in]) * 8
            mk = plsc.load_gather(tin.at[buf, 0], [pos_in])
            wk = plsc.load_gather(tin.at[buf, 1], [pos_in]) + T1
            m_idx = (mk >> 3) * (CP * 8) + it8 + (mk & 7)
            w_idx = (wk >> 3) * (CP * 8) + it8 + (wk & 7)
            pos = lane * 2 + j * (2 * LANES)
            plsc.store_scatter(idx2.at[buf], [pos], m_idx)
            plsc.store_scatter(idx2.at[buf], [pos + 1], w_idx)
            return carry2

        lax.fori_loop(0, CHUNK // LANES, vec_body, 0)

    for h in in_copies(0, 0):
        h.start()

    def loop_body(i, carry):
        for par in range(2):
            k = 2 * i + par
            buf = par
            nbuf = 1 - par

            @pl.when(k < NCHUNK)
            def _():
                @pl.when(k + 1 < NCHUNK)
                def _():
                    for h in in_copies(k + 1, nbuf):
                        h.start()

                for h in in_copies(k, buf):
                    h.wait()

                @pl.when(k >= 2)
                def _():
                    out_copy(k - 2, buf).wait()

                build_idx(buf)
                gather_copy(buf).start()

                @pl.when(k >= 1)
                def _():
                    gather_copy(nbuf).wait()
                    out_copy(k - 1, nbuf).start()

        return carry

    lax.fori_loop(0, (NCHUNK + 1) // 2, loop_body, 0)
    last = NCHUNK - 1
    gather_copy(last & 1).wait()
    out_copy(last, last & 1).start()
    out_copy(last - 1, 1 - (last & 1)).wait()
    out_copy(last, last & 1).wait()


def _tc_transpose_body(m_ref, w_ref, o_ref):
    o_ref[0] = m_ref[0:128, :].T
    o_ref[1] = jnp.concatenate([m_ref[128:192, :], w_ref[0:64, :]], axis=0).T
    o_ref[2, :, 0:16] = w_ref[64:80, :].T


_tc_transpose = pl.pallas_call(
    _tc_transpose_body,
    grid=(_TGRID,),
    in_specs=[
        pl.BlockSpec((192, _TW), lambda p: (0, p)),
        pl.BlockSpec((80, _TW), lambda p: (0, p)),
    ],
    out_specs=pl.BlockSpec((3, _TW, 128), lambda p: (0, p, 0)),
    out_shape=jax.ShapeDtypeStruct((3, CP, 128), jnp.float32),
)


def _tc_relayout_body(x_ref, o_ref):
    for i in range(8):
        z = x_ref[i].T
        for g in range(4):
            o_ref[i, :, pl.ds(g * 1024, 1024)] = z[32 * g:32 * (g + 1), :]


_tc_relayout = pl.pallas_call(
    _tc_relayout_body,
    grid=(L // 8,),
    in_specs=[pl.BlockSpec((8, 1024, 128), lambda l: (l, 0, 0))],
    out_specs=pl.BlockSpec((8, 2 * BASE, B), lambda l: (l, 0, 0)),
    out_shape=jax.ShapeDtypeStruct((L, 2 * BASE, B), jnp.float32),
)


def kernel(log_seqs, time1_seqs, time2_seqs, month_pop_table, week_pop_table):
    table = _tc_transpose(month_pop_table, week_pop_table).reshape(3 * CP * 8, BASE)
    t1 = time1_seqs.T.reshape(-1).astype(jnp.int32)
    t2 = time2_seqs.T.reshape(-1).astype(jnp.int32)
    item = log_seqs.T.reshape(-1).astype(jnp.int32)
    rows = _sc_gather(table, t1, t2, item)
    planes = _tc_relayout(rows.reshape(L, 1024, 128))
    return planes.transpose(2, 0, 1)


# speedup vs baseline: 2.8379x; 1.0065x over previous
import functools

import jax
import jax.numpy as jnp
from jax import lax
from jax.experimental import pallas as pl
from jax.experimental.pallas import tpu as pltpu
from jax.experimental.pallas import tpu_sc as plsc

B = 4096
L = 200
C = 100001
T1 = 12
BASE = 16
N = B * L

_TW = 4096
_TGRID = -(-C // _TW)
CP = _TGRID * _TW

_info = plsc.get_sparse_core_info()
NC, NS, LANES = _info.num_cores, _info.num_subcores, _info.num_lanes
NW = NC * NS
TOK_PER_W = N // NW
CHUNK = 1024
NCHUNK = TOK_PER_W // CHUNK


@functools.partial(
    pl.kernel,
    mesh=plsc.VectorSubcoreMesh(core_axis_name="c", subcore_axis_name="s"),
    out_type=jax.ShapeDtypeStruct((2 * N, BASE), jnp.float32),
    compiler_params=pltpu.CompilerParams(
        needs_layout_passes=False, use_tc_tiling_on_sc=False
    ),
    scratch_types=[
        pltpu.VMEM((2, 3, CHUNK), jnp.int32),
        pltpu.VMEM((2, 2 * CHUNK), jnp.int32),
        pltpu.VMEM((2, 2 * CHUNK, BASE), jnp.float32),
        pltpu.SemaphoreType.DMA,
        pltpu.SemaphoreType.DMA,
        pltpu.SemaphoreType.DMA,
        pltpu.SemaphoreType.DMA,
    ],
)
def _sc_gather(table_hbm, t1_hbm, t2_hbm, item_hbm, out_hbm,
               tin, idx2, rows2, sem_in0, sem_in1, sem_g, sem_out):
    wid = lax.axis_index("s") * NC + lax.axis_index("c")
    lane = lax.iota(jnp.int32, LANES)
    p0 = (lane & 3) * 256 + (lane >> 2)

    def in_copies(k, buf):
        cid = wid * NCHUNK + k
        pbase = (cid >> 2) * B + (cid & 3) * 256
        sem_in = sem_in1 if buf else sem_in0
        out = []
        for g in range(4):
            run = pl.ds(pbase + g * 1024, 256)
            for ai, src in enumerate((t1_hbm, t2_hbm, item_hbm)):
                out.append(pltpu.make_async_copy(
                    src.at[run], tin.at[buf, ai, pl.ds(g * 256, 256)], sem_in))
        return out

    def out_copy(k, buf):
        cid = wid * NCHUNK + k
        return pltpu.make_async_copy(
            rows2.at[buf], out_hbm.at[pl.ds(2 * cid * CHUNK, 2 * CHUNK)],
            sem_out)

    def gather_copy(buf):
        return pltpu.make_async_copy(
            table_hbm.at[idx2.at[buf]], rows2.at[buf], sem_g)

    def build_idx(buf):
        def vec_body(j, carry2):
            pos_in = p0 + 4 * j
            it8 = plsc.load_gather(tin.at[buf, 2], [pos_in]) * 8
            mk = plsc.load_gather(tin.at[buf, 0], [pos_in])
            wk = plsc.load_gather(tin.at[buf, 1], [pos_in]) + T1
            m_idx = (mk >> 3) * (CP * 8) + it8 + (mk & 7)
            w_idx = (wk >> 3) * (CP * 8) + it8 + (wk & 7)
            pos = lane * 2 + j * (2 * LANES)
            plsc.store_scatter(idx2.at[buf], [pos], m_idx)
            plsc.store_scatter(idx2.at[buf], [pos + 1], w_idx)
            return carry2

        lax.fori_loop(0, CHUNK // LANES, vec_body, 0)

    for h in in_copies(0, 0):
        h.start()

    def loop_body(i, carry):
        for par in range(2):
            k = 2 * i + par
            buf = par
            nbuf = 1 - par

            @pl.when(k < NCHUNK)
            def _():
                @pl.when(k + 1 < NCHUNK)
                def _():
                    for h in in_copies(k + 1, nbuf):
                        h.start()

                for h in in_copies(k, buf):
                    h.wait()

                @pl.when(k >= 2)
                def _():
                    out_copy(k - 2, buf).wait()

                build_idx(buf)
                gather_copy(buf).start()

                @pl.when(k >= 1)
                def _():
                    gather_copy(nbuf).wait()
                    out_copy(k - 1, nbuf).start()

        return carry

    lax.fori_loop(0, (NCHUNK + 1) // 2, loop_body, 0)
    last = NCHUNK - 1
    gather_copy(last & 1).wait()
    out_copy(last, last & 1).start()
    out_copy(last - 1, 1 - (last & 1)).wait()
    out_copy(last, last & 1).wait()


def _tc_transpose_body(m_ref, w_ref, o_ref):
    o_ref[0] = m_ref[0:128, :].T
    o_ref[1] = jnp.concatenate([m_ref[128:192, :], w_ref[0:64, :]], axis=0).T
    o_ref[2, :, 0:16] = w_ref[64:80, :].T


_tc_transpose = pl.pallas_call(
    _tc_transpose_body,
    grid=(_TGRID,),
    in_specs=[
        pl.BlockSpec((192, _TW), lambda p: (0, p)),
        pl.BlockSpec((80, _TW), lambda p: (0, p)),
    ],
    out_specs=pl.BlockSpec((3, _TW, 128), lambda p: (0, p, 0)),
    out_shape=jax.ShapeDtypeStruct((3, CP, 128), jnp.float32),
)


def _tc_relayout_body(x_ref, o_ref):
    for i in range(20):
        z = x_ref[i].T
        for g in range(4):
            o_ref[i, :, pl.ds(g * 1024, 1024)] = z[32 * g:32 * (g + 1), :]


_tc_relayout = pl.pallas_call(
    _tc_relayout_body,
    grid=(L // 20,),
    in_specs=[pl.BlockSpec((20, 1024, 128), lambda l: (l, 0, 0))],
    out_specs=pl.BlockSpec((20, 2 * BASE, B), lambda l: (l, 0, 0)),
    out_shape=jax.ShapeDtypeStruct((L, 2 * BASE, B), jnp.float32),
)


def kernel(log_seqs, time1_seqs, time2_seqs, month_pop_table, week_pop_table):
    table = _tc_transpose(month_pop_table, week_pop_table).reshape(3 * CP * 8, BASE)
    t1 = time1_seqs.T.reshape(-1).astype(jnp.int32)
    t2 = time2_seqs.T.reshape(-1).astype(jnp.int32)
    item = log_seqs.T.reshape(-1).astype(jnp.int32)
    rows = _sc_gather(table, t1, t2, item)
    planes = _tc_relayout(rows.reshape(L, 1024, 128))
    return planes.transpose(2, 0, 1)
